# Initial kernel scaffold; baseline (speedup 1.0000x reference)
import jax
import jax.numpy as jnp
from jax.experimental import pallas as pl


def kernel(x):
    raise NotImplementedError("write your pallas kernel here")


# speedup vs baseline: 38.4514x; 38.4514x over previous
import functools

import jax
import jax.numpy as jnp
from jax import lax
from jax.experimental import pallas as pl
from jax.experimental.pallas import tpu as pltpu
from jax.experimental.pallas import tpu_sc as plsc

B, S, C = 32, 32768, 64
K = 8
LANES = 16
NGROUPS = C // LANES
CHUNK = 512
NCHUNKS = S // CHUNK
RBLK = 8
NBLKS = CHUNK // RBLK

_info = plsc.get_sparse_core_info()
NC, NS = _info.num_cores, _info.num_subcores


def _insert(state, v):
    out = []
    t = v
    for j in range(K):
        mj = state[j]
        out.append(jnp.maximum(mj, t))
        t = jnp.minimum(mj, t)
    return tuple(out)


def _treemax(vs):
    while len(vs) > 1:
        vs = [jnp.maximum(vs[i], vs[i + 1]) for i in range(0, len(vs) - 1, 2)] + (
            [vs[-1]] if len(vs) % 2 else []
        )
    return vs[0]


@functools.partial(
    pl.kernel,
    mesh=plsc.VectorSubcoreMesh(core_axis_name="c", subcore_axis_name="s"),
    out_type=jax.ShapeDtypeStruct((B, K, C), jnp.float32),
    compiler_params=pltpu.CompilerParams(needs_layout_passes=False),
    scratch_types=[
        pltpu.VMEM((CHUNK, C), jnp.float32),
        pltpu.VMEM((K, C), jnp.float32),
    ],
)
def _topk_sc(x_hbm, out_hbm, buf, outb):
    cid = lax.axis_index("c")
    sid = lax.axis_index("s")
    b = sid * NC + cid

    init = tuple(
        jnp.full((LANES,), -jnp.inf, jnp.float32) for _ in range(K * NGROUPS)
    )

    def chunk_body(ci, state):
        pltpu.sync_copy(x_hbm.at[b, pl.ds(ci * CHUNK, CHUNK)], buf)

        def blk_body(bi, st):
            r0 = bi * RBLK
            new_st = []
            for g in range(NGROUPS):
                sg = st[g * K:(g + 1) * K]
                vs = [buf[r0 + r, pl.ds(g * LANES, LANES)] for r in range(RBLK)]
                bmax = _treemax(list(vs))
                cnt = plsc.all_reduce_population_count(bmax > sg[K - 1])
                pred = cnt[0] > 0

                def do(ops):
                    stt = tuple(ops[:K])
                    for v in ops[K:]:
                        stt = _insert(stt, v)
                    return stt

                def skip(ops):
                    return tuple(ops[:K])

                sg2 = lax.cond(pred, do, skip, tuple(sg) + tuple(vs))
                new_st.extend(sg2)
            return tuple(new_st)

        return lax.fori_loop(0, NBLKS, blk_body, state)

    final = lax.fori_loop(0, NCHUNKS, chunk_body, init)

    for g in range(NGROUPS):
        for j in range(K):
            outb[j, pl.ds(g * LANES, LANES)] = final[g * K + j]
    pltpu.sync_copy(outb, out_hbm.at[b])


def kernel(x):
    return _topk_sc(x)
